# baseline (speedup 1.0000x reference)
import jax
import jax.numpy as jnp
from jax import lax
from jax.experimental import pallas as pl
from jax.experimental.pallas import tpu as pltpu
from jax.experimental.pallas import tpu_sc as plsc

_B_ROWS = 4096
_SEQ = 200
_D = 128
_B = _B_ROWS * _SEQ
_NC = 2
_NS = 16
_NW = _NC * _NS
_BPW = _B // _NW
_NBUF = 5
_CHUNK = 128
_NCHUNK = _BPW // _CHUNK
_NG = _NCHUNK // _NBUF


def _emb_body(table_hbm, idx_hbm, out_hbm, idx_all, *bufs):
    wid = lax.axis_index("s") * _NC + lax.axis_index("c")
    base = pl.multiple_of(wid * _BPW, _BPW)
    rows = bufs[:_NBUF]
    sg = bufs[_NBUF:2 * _NBUF]
    so = bufs[2 * _NBUF:]

    def gather_start(chunk, b):
        off = pl.multiple_of(chunk * _CHUNK, _CHUNK)
        pltpu.async_copy(
            table_hbm.at[idx_all.at[pl.ds(off, _CHUNK)]], rows[b], sg[b])

    def gather_wait(b):
        pltpu.make_async_copy(
            table_hbm.at[idx_all.at[pl.ds(0, _CHUNK)]], rows[b], sg[b]).wait()

    def scatter_start(chunk, b):
        off = pl.multiple_of(base + chunk * _CHUNK, _CHUNK)
        return pltpu.async_copy(rows[b], out_hbm.at[pl.ds(off, _CHUNK)], so[b])

    head = _NBUF * _CHUNK
    pltpu.sync_copy(idx_hbm.at[pl.ds(base, head)], idx_all.at[pl.ds(0, head)])
    for b in range(_NBUF):
        gather_start(b, b)
    pltpu.sync_copy(idx_hbm.at[pl.ds(base + head, _BPW - head)],
                    idx_all.at[pl.ds(head, _BPW - head)])

    def body(g, carry):
        outs = []
        for b in range(_NBUF):
            gather_wait(b)
            outs.append(scatter_start(g * _NBUF + b, b))
        for b in range(_NBUF):
            outs[b].wait()
            gather_start((g + 1) * _NBUF + b, b)
        return carry

    lax.fori_loop(0, _NG - 1, body, 0)

    outs = []
    for b in range(_NBUF):
        gather_wait(b)
        outs.append(scatter_start((_NG - 1) * _NBUF + b, b))
    for o in outs:
        o.wait()


@jax.jit
def kernel(y, emb_weight):
    yf = y.reshape(_B)
    mesh = plsc.VectorSubcoreMesh(core_axis_name="c", subcore_axis_name="s")
    k = pl.kernel(
        _emb_body,
        out_type=jax.ShapeDtypeStruct((_B, _D), jnp.float32),
        mesh=mesh,
        scratch_types=(
            [pltpu.VMEM((_BPW,), jnp.int32)]
            + [pltpu.VMEM((_CHUNK, _D), jnp.float32)] * _NBUF
            + [pltpu.SemaphoreType.DMA] * (2 * _NBUF)
        ),
    )
    out = k(emb_weight, yf)
    return out.reshape(_B_ROWS, _SEQ, _D)


# speedup vs baseline: 1.0031x; 1.0031x over previous
import jax
import jax.numpy as jnp
from jax import lax
from jax.experimental import pallas as pl
from jax.experimental.pallas import tpu as pltpu
from jax.experimental.pallas import tpu_sc as plsc

_B_ROWS = 4096
_SEQ = 200
_D = 128
_B = _B_ROWS * _SEQ
_NC = 2
_NS = 16
_NW = _NC * _NS
_BPW = _B // _NW
_TURN = 400
_SUB = 200
_NSUB = _TURN // _SUB
_NTURN = _BPW // _TURN


def _emb_body(table_hbm, idx_hbm, out_hbm, idx_all,
              big0, big1, sg0, sg1, so0, so1):
    wid = lax.axis_index("s") * _NC + lax.axis_index("c")
    base = pl.multiple_of(wid * _BPW, _BPW)
    big = (big0, big1)
    sg = (sg0, sg1)
    so = (so0, so1)

    def gathers_start(t, b):
        for j in range(_NSUB):
            off = pl.multiple_of(t * _TURN + j * _SUB, 8)
            pltpu.async_copy(
                table_hbm.at[idx_all.at[pl.ds(off, _SUB)]],
                big[b].at[pl.ds(j * _SUB, _SUB), :], sg[b])

    def gathers_wait(b):
        pltpu.make_async_copy(
            table_hbm.at[idx_all.at[pl.ds(0, _TURN)]], big[b], sg[b]).wait()

    def scatter_start(t, b):
        off = pl.multiple_of(base + t * _TURN, 8)
        pltpu.async_copy(big[b], out_hbm.at[pl.ds(off, _TURN)], so[b])

    def scatter_wait(b):
        pltpu.make_async_copy(
            big[b], out_hbm.at[pl.ds(base, _TURN)], so[b]).wait()

    pltpu.sync_copy(idx_hbm.at[pl.ds(base, _BPW)], idx_all)

    gathers_start(0, 0)
    gathers_wait(0)
    scatter_start(0, 0)
    gathers_start(1, 1)

    def body(g, carry):
        t1 = 2 * g + 1
        gathers_wait(1)
        scatter_start(t1, 1)
        scatter_wait(0)
        gathers_start(t1 + 1, 0)
        gathers_wait(0)
        scatter_start(t1 + 1, 0)
        scatter_wait(1)
        gathers_start(t1 + 2, 1)
        return carry

    lax.fori_loop(0, (_NTURN - 2) // 2, body, 0)

    gathers_wait(1)
    scatter_start(_NTURN - 1, 1)
    scatter_wait(0)
    scatter_wait(1)


@jax.jit
def kernel(y, emb_weight):
    yf = y.reshape(_B)
    mesh = plsc.VectorSubcoreMesh(core_axis_name="c", subcore_axis_name="s")
    k = pl.kernel(
        _emb_body,
        out_type=jax.ShapeDtypeStruct((_B, _D), jnp.float32),
        mesh=mesh,
        scratch_types=(
            [pltpu.VMEM((_BPW,), jnp.int32)]
            + [pltpu.VMEM((_TURN, _D), jnp.float32)] * 2
            + [pltpu.SemaphoreType.DMA] * 4
        ),
    )
    out = k(emb_weight, yf)
    return out.reshape(_B_ROWS, _SEQ, _D)
